# baseline (speedup 1.0000x reference)
import functools

import jax
import jax.numpy as jnp
from jax import lax
from jax.experimental import pallas as pl
from jax.experimental.pallas import tpu as pltpu
from jax.experimental.pallas import tpu_sc as plsc

VOCAB = 1000
EMBED = 128
UNITS = 512
B = 64
T = 128
NTOK = B * T
UNROLL = 8

_SC_CHUNK = 128


def _make_sc_gather(n_rows: int, dim: int):
    info = plsc.get_sparse_core_info()
    nw = info.num_cores * info.num_subcores
    rows_per_w = n_rows // nw
    n_chunks = rows_per_w // _SC_CHUNK
    mesh = plsc.VectorSubcoreMesh(core_axis_name="c", subcore_axis_name="s")

    @functools.partial(
        pl.kernel,
        mesh=mesh,
        out_type=jax.ShapeDtypeStruct((n_rows, dim), jnp.float32),
        scratch_types=[
            pltpu.VMEM((_SC_CHUNK,), jnp.int32),
            pltpu.VMEM((_SC_CHUNK, dim), jnp.float32),
            pltpu.SemaphoreType.DMA,
        ],
    )
    def sc_gather(table_hbm, idx_hbm, out_hbm, idx_v, rows_v, sem):
        wid = lax.axis_index("s") * info.num_cores + lax.axis_index("c")
        base = wid * rows_per_w
        for ch in range(n_chunks):
            off = base + ch * _SC_CHUNK
            pltpu.sync_copy(idx_hbm.at[pl.ds(off, _SC_CHUNK)], idx_v)
            pltpu.async_copy(table_hbm.at[idx_v], rows_v, sem).wait()
            pltpu.sync_copy(rows_v, out_hbm.at[pl.ds(off, _SC_CHUNK)])

    return sc_gather


def _fused_step(
    xe_ref, k_ref, r_ref, b_ref, wd_ref, bd_ref, out_ref, h_ref, c_ref, hs_ref
):
    i = pl.program_id(0)

    @pl.when(i == 0)
    def _():
        h_ref[...] = jnp.zeros_like(h_ref)
        c_ref[...] = jnp.zeros_like(c_ref)

    xz_all = jnp.dot(
        xe_ref[...].reshape(UNROLL * B, EMBED).astype(jnp.bfloat16),
        k_ref[...],
        preferred_element_type=jnp.float32,
    ) + b_ref[...]
    for j in range(UNROLL):
        h = h_ref[...].astype(jnp.bfloat16)
        z = xz_all[j * B : (j + 1) * B] + jnp.dot(
            h, r_ref[...], preferred_element_type=jnp.float32
        )
        ig = jax.nn.sigmoid(z[:, :UNITS])
        fg = jax.nn.sigmoid(z[:, UNITS : 2 * UNITS])
        gg = jnp.tanh(z[:, 2 * UNITS : 3 * UNITS])
        og = jax.nn.sigmoid(z[:, 3 * UNITS :])
        c = fg * c_ref[...] + ig * gg
        h_new = og * jnp.tanh(c)
        c_ref[...] = c
        h_ref[...] = h_new
        hs_ref[pl.ds(j * B, B), :] = h_new

    y = jnp.dot(
        hs_ref[...].astype(jnp.bfloat16),
        wd_ref[...],
        preferred_element_type=jnp.float32,
    ) + bd_ref[...]
    for j in range(UNROLL):
        out_ref[:, j, :] = y[j * B : (j + 1) * B]


def _lstm_fused(x_emb, kernel_w, rec_kernel, bias2d, W_dense, bd2d, *, interpret=False):
    return pl.pallas_call(
        _fused_step,
        grid=(T // UNROLL,),
        in_specs=[
            pl.BlockSpec((UNROLL, B, EMBED), lambda i: (i, 0, 0)),
            pl.BlockSpec((EMBED, 4 * UNITS), lambda i: (0, 0)),
            pl.BlockSpec((UNITS, 4 * UNITS), lambda i: (0, 0)),
            pl.BlockSpec((1, 4 * UNITS), lambda i: (0, 0)),
            pl.BlockSpec((UNITS, VOCAB), lambda i: (0, 0)),
            pl.BlockSpec((1, VOCAB), lambda i: (0, 0)),
        ],
        out_specs=pl.BlockSpec((B, UNROLL, VOCAB), lambda i: (0, i, 0)),
        out_shape=jax.ShapeDtypeStruct((B, T, VOCAB), jnp.float32),
        scratch_shapes=[
            pltpu.VMEM((B, UNITS), jnp.float32),
            pltpu.VMEM((B, UNITS), jnp.float32),
            pltpu.VMEM((UNROLL * B, UNITS), jnp.float32),
        ],
        interpret=interpret,
    )(x_emb, kernel_w, rec_kernel, bias2d, W_dense, bd2d)


def kernel(inputs, W_emb, kernel, rec_kernel, bias, W_dense, b_dense):
    idx = inputs.T.reshape(-1).astype(jnp.int32)
    x_emb = _make_sc_gather(NTOK, EMBED)(W_emb, idx)
    logits = _lstm_fused(
        x_emb.reshape(T, B, EMBED),
        kernel.astype(jnp.bfloat16),
        rec_kernel.astype(jnp.bfloat16),
        bias.reshape(1, 4 * UNITS),
        W_dense.astype(jnp.bfloat16),
        b_dense.reshape(1, VOCAB),
    )
    return logits


# speedup vs baseline: 1.1102x; 1.1102x over previous
import functools

import jax
import jax.numpy as jnp
from jax import lax
from jax.experimental import pallas as pl
from jax.experimental.pallas import tpu as pltpu
from jax.experimental.pallas import tpu_sc as plsc

VOCAB = 1000
EMBED = 128
UNITS = 512
B = 64
T = 128
NTOK = B * T
UNROLL = 8

_SC_CHUNK = 128


def _make_sc_gather(n_rows: int, dim: int):
    info = plsc.get_sparse_core_info()
    nw = info.num_cores * info.num_subcores
    rows_per_w = n_rows // nw
    n_chunks = rows_per_w // _SC_CHUNK
    mesh = plsc.VectorSubcoreMesh(core_axis_name="c", subcore_axis_name="s")

    @functools.partial(
        pl.kernel,
        mesh=mesh,
        out_type=jax.ShapeDtypeStruct((n_rows, dim), jnp.float32),
        scratch_types=[
            pltpu.VMEM((_SC_CHUNK,), jnp.int32),
            pltpu.VMEM((_SC_CHUNK, dim), jnp.float32),
            pltpu.SemaphoreType.DMA,
        ],
    )
    def sc_gather(table_hbm, idx_hbm, out_hbm, idx_v, rows_v, sem):
        wid = lax.axis_index("s") * info.num_cores + lax.axis_index("c")
        base = wid * rows_per_w
        for ch in range(n_chunks):
            off = base + ch * _SC_CHUNK
            pltpu.sync_copy(idx_hbm.at[pl.ds(off, _SC_CHUNK)], idx_v)
            pltpu.async_copy(table_hbm.at[idx_v], rows_v, sem).wait()
            pltpu.sync_copy(rows_v, out_hbm.at[pl.ds(off, _SC_CHUNK)])

    return sc_gather


def _fused_step(
    xe_ref, k_ref, r_ref, b_ref, wd_ref, bd_ref, out_ref, h_ref, c_ref, hs_ref
):
    i = pl.program_id(0)

    @pl.when(i == 0)
    def _():
        h_ref[...] = jnp.zeros_like(h_ref)
        c_ref[...] = jnp.zeros_like(c_ref)

    xz_all = jnp.dot(
        xe_ref[...].reshape(UNROLL * B, EMBED),
        k_ref[...],
        preferred_element_type=jnp.float32,
    ) + b_ref[...]
    for j in range(UNROLL):
        z = xz_all[j * B : (j + 1) * B] + jnp.dot(
            h_ref[...], r_ref[...], preferred_element_type=jnp.float32
        )
        ig = jax.nn.sigmoid(z[:, :UNITS])
        fg = jax.nn.sigmoid(z[:, UNITS : 2 * UNITS])
        gg = jnp.tanh(z[:, 2 * UNITS : 3 * UNITS])
        og = jax.nn.sigmoid(z[:, 3 * UNITS :])
        c = fg * c_ref[...] + ig * gg
        h_new = og * jnp.tanh(c)
        c_ref[...] = c
        h_ref[...] = h_new
        hs_ref[pl.ds(j * B, B), :] = h_new

    out_ref[...] = (
        jnp.dot(
            hs_ref[...],
            wd_ref[...],
            preferred_element_type=jnp.float32,
        )
        + bd_ref[...]
    )


def _lstm_fused(x_emb, kernel_w, rec_kernel, bias2d, W_dense, bd2d, *, interpret=False):
    return pl.pallas_call(
        _fused_step,
        grid=(T // UNROLL,),
        in_specs=[
            pl.BlockSpec((UNROLL, B, EMBED), lambda i: (i, 0, 0)),
            pl.BlockSpec((EMBED, 4 * UNITS), lambda i: (0, 0)),
            pl.BlockSpec((UNITS, 4 * UNITS), lambda i: (0, 0)),
            pl.BlockSpec((1, 4 * UNITS), lambda i: (0, 0)),
            pl.BlockSpec((UNITS, VOCAB), lambda i: (0, 0)),
            pl.BlockSpec((1, VOCAB), lambda i: (0, 0)),
        ],
        out_specs=pl.BlockSpec((UNROLL * B, VOCAB), lambda i: (i, 0)),
        out_shape=jax.ShapeDtypeStruct((NTOK, VOCAB), jnp.float32),
        scratch_shapes=[
            pltpu.VMEM((B, UNITS), jnp.float32),
            pltpu.VMEM((B, UNITS), jnp.float32),
            pltpu.VMEM((UNROLL * B, UNITS), jnp.float32),
        ],
        interpret=interpret,
    )(x_emb, kernel_w, rec_kernel, bias2d, W_dense, bd2d)


def kernel(inputs, W_emb, kernel, rec_kernel, bias, W_dense, b_dense):
    idx = inputs.T.reshape(-1).astype(jnp.int32)
    x_emb = _make_sc_gather(NTOK, EMBED)(W_emb, idx)
    logits = _lstm_fused(
        x_emb.reshape(T, B, EMBED),
        kernel,
        rec_kernel,
        bias.reshape(1, 4 * UNITS),
        W_dense,
        b_dense.reshape(1, VOCAB),
    )
    return logits.reshape(T, B, VOCAB).transpose(1, 0, 2)
